# Initial kernel scaffold; baseline (speedup 1.0000x reference)
import functools

import jax
import jax.numpy as jnp
from jax import lax
from jax.experimental import pallas as pl
from jax.experimental.pallas import tpu as pltpu
from jax.experimental.pallas import tpu_sc as plsc

NU = 5000
NI = 5000
NN = NU + NI
NE = 160000
FD = 128
FD3 = 3 * FD
EPS = 0.1

NS = 16
EPT = NE // NS
CH = 80
NCHUNK = EPT // CH

ZCH = 80
NZFULL = NU // ZCH
ZREM = NU - NZFULL * ZCH
ZREM_TILE = NZFULL % NS

_F32 = jnp.float32


@functools.partial(
    pl.kernel,
    out_type=jax.ShapeDtypeStruct((NN,), _F32),
    mesh=plsc.VectorSubcoreMesh(core_axis_name="c", subcore_axis_name="s"),
    scratch_types=[
        pltpu.VMEM((NCHUNK, CH), jnp.int32),
        pltpu.VMEM((CH,), _F32),
        pltpu.VMEM((320,), _F32),
        pltpu.VMEM_SHARED((NU,), _F32),
    ],
)
def _deg_kernel(eidx_hbm, deg_hbm, idx_v, ones_v, zrow_v, acc):
    c = lax.axis_index("c")
    s = lax.axis_index("s")
    pltpu.sync_copy(eidx_hbm.at[c, s], idx_v)

    @pl.loop(0, CH // 16)
    def _(i):
        ones_v[pl.ds(i * 16, 16)] = jnp.ones((16,), _F32)

    @pl.loop(0, 320 // 16)
    def _(i):
        zrow_v[pl.ds(i * 16, 16)] = jnp.zeros((16,), _F32)

    @pl.when(s < 15)
    def _():
        pltpu.sync_copy(zrow_v, acc.at[pl.ds(s * 320, 320)])

    @pl.when(s == 15)
    def _():
        pltpu.sync_copy(zrow_v.at[pl.ds(0, 200)], acc.at[pl.ds(4800, 200)])

    plsc.subcore_barrier()

    @pl.loop(0, NCHUNK)
    def _(j):
        pltpu.sync_copy(ones_v, acc.at[idx_v.at[j]], add=True)

    plsc.subcore_barrier()

    @pl.when(s < 15)
    def _():
        pltpu.sync_copy(acc.at[pl.ds(s * 320, 320)],
                        deg_hbm.at[pl.ds(c * NU + s * 320, 320)])

    @pl.when(s == 15)
    def _():
        pltpu.sync_copy(acc.at[pl.ds(4800, 200)],
                        deg_hbm.at[pl.ds(c * NU + 4800, 200)])


@functools.partial(
    pl.kernel,
    out_type=jax.ShapeDtypeStruct((NN, FD3), _F32),
    mesh=plsc.VectorSubcoreMesh(core_axis_name="c", subcore_axis_name="s"),
    scratch_types=[
        pltpu.VMEM((NCHUNK, CH), jnp.int32),
        pltpu.VMEM((NCHUNK, CH), jnp.int32),
        pltpu.VMEM((CH, FD3), _F32),
        pltpu.VMEM((ZCH, FD3), _F32),
        pltpu.VMEM_SHARED((NU, FD3), _F32),
        pltpu.SemaphoreType.DMA,
    ],
)
def _spmm_kernel(feat_hbm, srcidx_hbm, dstidx_hbm, out_hbm,
                 sidx_v, didx_v, buf_v, zbuf_v, acc, sem):
    c = lax.axis_index("c")
    s = lax.axis_index("s")
    pltpu.sync_copy(srcidx_hbm.at[c, s], sidx_v)
    pltpu.sync_copy(dstidx_hbm.at[c, s], didx_v)

    @pl.loop(0, ZCH)
    def _(i):
        @pl.loop(0, FD3 // 16)
        def _(j):
            zbuf_v[i, pl.ds(j * 16, 16)] = jnp.zeros((16,), _F32)

    @pl.loop(0, NZFULL)
    def _(k):
        @pl.when(lax.rem(k, NS) == s)
        def _():
            pltpu.sync_copy(zbuf_v, acc.at[pl.ds(k * ZCH, ZCH)])

    @pl.when(s == ZREM_TILE)
    def _():
        pltpu.sync_copy(zbuf_v.at[pl.ds(0, ZREM)],
                        acc.at[pl.ds(NZFULL * ZCH, ZREM)])

    plsc.subcore_barrier()

    @pl.loop(0, NCHUNK)
    def _(j):
        pltpu.async_copy(feat_hbm.at[sidx_v.at[j]], buf_v, sem).wait()
        pltpu.sync_copy(buf_v, acc.at[didx_v.at[j]], add=True)

    plsc.subcore_barrier()

    @pl.loop(0, NZFULL)
    def _(k):
        @pl.when(lax.rem(k, NS) == s)
        def _():
            pltpu.sync_copy(acc.at[pl.ds(k * ZCH, ZCH)],
                            out_hbm.at[pl.ds(c * NU + k * ZCH, ZCH)])

    @pl.when(s == ZREM_TILE)
    def _():
        pltpu.sync_copy(acc.at[pl.ds(NZFULL * ZCH, ZREM)],
                        out_hbm.at[pl.ds(c * NU + NZFULL * ZCH, ZREM)])


_RB = 2000


def _prep_body(deg_ref, x_ref, r1_ref, r2_ref, feat_ref):
    deg = deg_ref[...]
    d_inv = jnp.where(deg > 0.0, lax.rsqrt(deg), 0.0)
    xb = x_ref[...]
    sx = jnp.sign(xb)
    p1 = xb + r1_ref[...] * sx * EPS
    p2 = xb + r2_ref[...] * sx * EPS
    feat_ref[:, 0:FD] = xb * d_inv
    feat_ref[:, FD:2 * FD] = p1 * d_inv
    feat_ref[:, 2 * FD:3 * FD] = p2 * d_inv


_prep = pl.pallas_call(
    _prep_body,
    grid=(NN // _RB,),
    in_specs=[
        pl.BlockSpec((_RB, 1), lambda i: (i, 0)),
        pl.BlockSpec((_RB, FD), lambda i: (i, 0)),
        pl.BlockSpec((_RB, FD), lambda i: (i, 0)),
        pl.BlockSpec((_RB, FD), lambda i: (i, 0)),
    ],
    out_specs=pl.BlockSpec((_RB, FD3), lambda i: (i, 0)),
    out_shape=jax.ShapeDtypeStruct((NN, FD3), _F32),
)


def _post_body(deg_ref, x_ref, r1_ref, r2_ref, acc_ref,
               alle_ref, allp1_ref, allp2_ref, g0_ref):
    deg = deg_ref[...]
    d_inv = jnp.where(deg > 0.0, lax.rsqrt(deg), 0.0)
    xb = x_ref[...]
    sx = jnp.sign(xb)
    p1 = xb + r1_ref[...] * sx * EPS
    p2 = xb + r2_ref[...] * sx * EPS
    g0 = acc_ref[:, 0:FD] * d_inv
    g1 = acc_ref[:, FD:2 * FD] * d_inv
    g2 = acc_ref[:, 2 * FD:3 * FD] * d_inv
    alle_ref[...] = 2.0 * xb + g0
    allp1_ref[...] = 2.0 * p1 + g1
    allp2_ref[...] = 2.0 * p2 + g2
    g0_ref[...] = g0


_post = pl.pallas_call(
    _post_body,
    grid=(NN // _RB,),
    in_specs=[
        pl.BlockSpec((_RB, 1), lambda i: (i, 0)),
        pl.BlockSpec((_RB, FD), lambda i: (i, 0)),
        pl.BlockSpec((_RB, FD), lambda i: (i, 0)),
        pl.BlockSpec((_RB, FD), lambda i: (i, 0)),
        pl.BlockSpec((_RB, FD3), lambda i: (i, 0)),
    ],
    out_specs=[
        pl.BlockSpec((_RB, FD), lambda i: (i, 0)),
        pl.BlockSpec((_RB, FD), lambda i: (i, 0)),
        pl.BlockSpec((_RB, FD), lambda i: (i, 0)),
        pl.BlockSpec((_RB, FD), lambda i: (i, 0)),
    ],
    out_shape=[
        jax.ShapeDtypeStruct((NN, FD), _F32),
        jax.ShapeDtypeStruct((NN, FD), _F32),
        jax.ShapeDtypeStruct((NN, FD), _F32),
        jax.ShapeDtypeStruct((NN, FD), _F32),
    ],
)


def kernel(x, rand1, rand2, edge_u, edge_i):
    eu = edge_u.astype(jnp.int32)
    ei = edge_i.astype(jnp.int32)
    dst_idx = jnp.stack([eu, ei]).reshape(2, NS, NCHUNK, CH)
    src_idx = jnp.stack([ei + NU, eu]).reshape(2, NS, NCHUNK, CH)

    deg = _deg_kernel(dst_idx)
    deg2 = deg.reshape(NN, 1)
    feat = _prep(deg2, x, rand1, rand2)
    accf = _spmm_kernel(feat, src_idx, dst_idx)
    all_e, all_p1, all_p2, g0 = _post(deg2, x, rand1, rand2, accf)
    return (all_e[:NU], all_e[NU:], all_p1[:NU], all_p1[NU:],
            all_p2[:NU], all_p2[NU:], g0)


# speedup vs baseline: 17.1795x; 17.1795x over previous
import functools

import jax
import jax.numpy as jnp
from jax import lax
from jax.experimental import pallas as pl
from jax.experimental.pallas import tpu as pltpu
from jax.experimental.pallas import tpu_sc as plsc

NU = 5000
NI = 5000
NN = NU + NI
NE = 160000
FD = 128
EPS = 0.1

NS = 16
EPT = NE // NS
CH = 80
NCHUNK = EPT // CH

ZCH = 80
NZFULL = NU // ZCH
ZREM = NU - NZFULL * ZCH
ZREM_TILE = NZFULL % NS

_F32 = jnp.float32


@functools.partial(
    pl.kernel,
    out_type=jax.ShapeDtypeStruct((NN,), _F32),
    mesh=plsc.VectorSubcoreMesh(core_axis_name="c", subcore_axis_name="s"),
    scratch_types=[
        pltpu.VMEM((NCHUNK, CH), jnp.int32),
        pltpu.VMEM((CH,), _F32),
        pltpu.VMEM((320,), _F32),
        pltpu.VMEM_SHARED((NU,), _F32),
    ],
)
def _deg_kernel(eidx_hbm, deg_hbm, idx_v, ones_v, zrow_v, acc):
    c = lax.axis_index("c")
    s = lax.axis_index("s")
    pltpu.sync_copy(eidx_hbm.at[c, s], idx_v)

    @pl.loop(0, CH // 16)
    def _(i):
        ones_v[pl.ds(i * 16, 16)] = jnp.ones((16,), _F32)

    @pl.loop(0, 320 // 16)
    def _(i):
        zrow_v[pl.ds(i * 16, 16)] = jnp.zeros((16,), _F32)

    @pl.when(s < 15)
    def _():
        pltpu.sync_copy(zrow_v, acc.at[pl.ds(s * 320, 320)])

    @pl.when(s == 15)
    def _():
        pltpu.sync_copy(zrow_v.at[pl.ds(0, 200)], acc.at[pl.ds(4800, 200)])

    plsc.subcore_barrier()

    @pl.loop(0, NCHUNK)
    def _(j):
        pltpu.sync_copy(ones_v, acc.at[idx_v.at[j]], add=True)

    plsc.subcore_barrier()

    @pl.when(s < 15)
    def _():
        pltpu.sync_copy(acc.at[pl.ds(s * 320, 320)], zrow_v)
        pltpu.sync_copy(zrow_v, deg_hbm.at[pl.ds(c * NU + s * 320, 320)])

    @pl.when(s == 15)
    def _():
        pltpu.sync_copy(acc.at[pl.ds(4800, 200)], zrow_v.at[pl.ds(0, 200)])
        pltpu.sync_copy(zrow_v.at[pl.ds(0, 200)],
                        deg_hbm.at[pl.ds(c * NU + 4800, 200)])


_OUT3 = [jax.ShapeDtypeStruct((NN, FD), _F32) for _ in range(3)]


@functools.partial(
    pl.kernel,
    out_type=_OUT3,
    mesh=plsc.VectorSubcoreMesh(core_axis_name="c", subcore_axis_name="s"),
    scratch_types=[
        pltpu.VMEM((NCHUNK, CH), jnp.int32),
        pltpu.VMEM((NCHUNK, CH), jnp.int32),
        pltpu.VMEM((CH, FD), _F32),
        pltpu.VMEM((CH, FD), _F32),
        pltpu.VMEM((ZCH, FD), _F32),
        pltpu.VMEM_SHARED((NU, FD), _F32),
        pltpu.SemaphoreType.DMA,
        pltpu.SemaphoreType.DMA,
    ],
)
def _spmm_kernel(f0_hbm, f1_hbm, f2_hbm, srcidx_hbm, dstidx_hbm,
                 o0_hbm, o1_hbm, o2_hbm,
                 sidx_v, didx_v, buf0_v, buf1_v, zbuf_v, acc, sem0, sem1):
    c = lax.axis_index("c")
    s = lax.axis_index("s")
    pltpu.sync_copy(srcidx_hbm.at[c, s], sidx_v)
    pltpu.sync_copy(dstidx_hbm.at[c, s], didx_v)

    @pl.loop(0, ZCH)
    def _(i):
        @pl.loop(0, FD // 16)
        def _(j):
            zbuf_v[i, pl.ds(j * 16, 16)] = jnp.zeros((16,), _F32)

    for feat_hbm, out_hbm in ((f0_hbm, o0_hbm), (f1_hbm, o1_hbm),
                              (f2_hbm, o2_hbm)):
        @pl.loop(0, NZFULL)
        def _(k):
            @pl.when(lax.rem(k, NS) == s)
            def _():
                pltpu.sync_copy(zbuf_v, acc.at[pl.ds(k * ZCH, ZCH)])

        @pl.when(s == ZREM_TILE)
        def _():
            pltpu.sync_copy(zbuf_v.at[pl.ds(0, ZREM)],
                            acc.at[pl.ds(NZFULL * ZCH, ZREM)])

        plsc.subcore_barrier()

        pltpu.async_copy(feat_hbm.at[sidx_v.at[0]], buf0_v, sem0)

        @pl.loop(0, NCHUNK // 2)
        def _(k):
            a = 2 * k
            b = a + 1
            pltpu.make_async_copy(feat_hbm.at[sidx_v.at[a]], buf0_v,
                                  sem0).wait()
            pltpu.async_copy(feat_hbm.at[sidx_v.at[b]], buf1_v, sem1)
            pltpu.sync_copy(buf0_v, acc.at[didx_v.at[a]], add=True)
            pltpu.make_async_copy(feat_hbm.at[sidx_v.at[b]], buf1_v,
                                  sem1).wait()
            pltpu.async_copy(feat_hbm.at[sidx_v.at[a + 2]], buf0_v, sem0)
            pltpu.sync_copy(buf1_v, acc.at[didx_v.at[b]], add=True)

        pltpu.make_async_copy(feat_hbm.at[sidx_v.at[NCHUNK - 1]], buf0_v,
                              sem0).wait()
        pltpu.sync_copy(buf0_v, acc.at[didx_v.at[NCHUNK - 1]], add=True)

        plsc.subcore_barrier()

        @pl.loop(0, NZFULL)
        def _(k):
            @pl.when(lax.rem(k, NS) == s)
            def _():
                pltpu.sync_copy(acc.at[pl.ds(k * ZCH, ZCH)], buf0_v)
                pltpu.sync_copy(buf0_v,
                                out_hbm.at[pl.ds(c * NU + k * ZCH, ZCH)])

        @pl.when(s == ZREM_TILE)
        def _():
            pltpu.sync_copy(acc.at[pl.ds(NZFULL * ZCH, ZREM)],
                            buf0_v.at[pl.ds(0, ZREM)])
            pltpu.sync_copy(buf0_v.at[pl.ds(0, ZREM)],
                            out_hbm.at[pl.ds(c * NU + NZFULL * ZCH, ZREM)])

        plsc.subcore_barrier()


_RB = 2000


def _prep_body(deg_ref, x_ref, r1_ref, r2_ref, f0_ref, f1_ref, f2_ref):
    deg = deg_ref[...]
    d_inv = jnp.where(deg > 0.0, lax.rsqrt(deg), 0.0)
    xb = x_ref[...]
    sx = jnp.sign(xb)
    p1 = xb + r1_ref[...] * sx * EPS
    p2 = xb + r2_ref[...] * sx * EPS
    f0_ref[...] = xb * d_inv
    f1_ref[...] = p1 * d_inv
    f2_ref[...] = p2 * d_inv


_prep = pl.pallas_call(
    _prep_body,
    grid=(NN // _RB,),
    in_specs=[
        pl.BlockSpec((_RB, 1), lambda i: (i, 0)),
        pl.BlockSpec((_RB, FD), lambda i: (i, 0)),
        pl.BlockSpec((_RB, FD), lambda i: (i, 0)),
        pl.BlockSpec((_RB, FD), lambda i: (i, 0)),
    ],
    out_specs=[
        pl.BlockSpec((_RB, FD), lambda i: (i, 0)),
        pl.BlockSpec((_RB, FD), lambda i: (i, 0)),
        pl.BlockSpec((_RB, FD), lambda i: (i, 0)),
    ],
    out_shape=[
        jax.ShapeDtypeStruct((NN, FD), _F32),
        jax.ShapeDtypeStruct((NN, FD), _F32),
        jax.ShapeDtypeStruct((NN, FD), _F32),
    ],
)


def _post_body(deg_ref, x_ref, r1_ref, r2_ref, a0_ref, a1_ref, a2_ref,
               alle_ref, allp1_ref, allp2_ref, g0_ref):
    deg = deg_ref[...]
    d_inv = jnp.where(deg > 0.0, lax.rsqrt(deg), 0.0)
    xb = x_ref[...]
    sx = jnp.sign(xb)
    p1 = xb + r1_ref[...] * sx * EPS
    p2 = xb + r2_ref[...] * sx * EPS
    g0 = a0_ref[...] * d_inv
    g1 = a1_ref[...] * d_inv
    g2 = a2_ref[...] * d_inv
    alle_ref[...] = 2.0 * xb + g0
    allp1_ref[...] = 2.0 * p1 + g1
    allp2_ref[...] = 2.0 * p2 + g2
    g0_ref[...] = g0


_post = pl.pallas_call(
    _post_body,
    grid=(NN // _RB,),
    in_specs=[
        pl.BlockSpec((_RB, 1), lambda i: (i, 0)),
        pl.BlockSpec((_RB, FD), lambda i: (i, 0)),
        pl.BlockSpec((_RB, FD), lambda i: (i, 0)),
        pl.BlockSpec((_RB, FD), lambda i: (i, 0)),
        pl.BlockSpec((_RB, FD), lambda i: (i, 0)),
        pl.BlockSpec((_RB, FD), lambda i: (i, 0)),
        pl.BlockSpec((_RB, FD), lambda i: (i, 0)),
    ],
    out_specs=[
        pl.BlockSpec((_RB, FD), lambda i: (i, 0)),
        pl.BlockSpec((_RB, FD), lambda i: (i, 0)),
        pl.BlockSpec((_RB, FD), lambda i: (i, 0)),
        pl.BlockSpec((_RB, FD), lambda i: (i, 0)),
    ],
    out_shape=[
        jax.ShapeDtypeStruct((NN, FD), _F32),
        jax.ShapeDtypeStruct((NN, FD), _F32),
        jax.ShapeDtypeStruct((NN, FD), _F32),
        jax.ShapeDtypeStruct((NN, FD), _F32),
    ],
)


def kernel(x, rand1, rand2, edge_u, edge_i):
    eu = edge_u.astype(jnp.int32)
    ei = edge_i.astype(jnp.int32)
    dst_idx = jnp.stack([eu, ei]).reshape(2, NS, NCHUNK, CH)
    src_idx = jnp.stack([ei + NU, eu]).reshape(2, NS, NCHUNK, CH)

    deg = _deg_kernel(dst_idx)
    deg2 = deg.reshape(NN, 1)
    f0, f1, f2 = _prep(deg2, x, rand1, rand2)
    a0, a1, a2 = _spmm_kernel(f0, f1, f2, src_idx, dst_idx)
    all_e, all_p1, all_p2, g0 = _post(deg2, x, rand1, rand2, a0, a1, a2)
    return (all_e[:NU], all_e[NU:], all_p1[:NU], all_p1[NU:],
            all_p2[:NU], all_p2[NU:], g0)


# speedup vs baseline: 20.3451x; 1.1843x over previous
import functools

import jax
import jax.numpy as jnp
from jax import lax
from jax.experimental import pallas as pl
from jax.experimental.pallas import tpu as pltpu
from jax.experimental.pallas import tpu_sc as plsc

NU = 5000
NI = 5000
NN = NU + NI
NE = 160000
FD = 128
EPS = 0.1

NS = 16
EPT = NE // NS
CH = 125
NCHUNK = EPT // CH

ZCH = 80
NZFULL = NU // ZCH
ZREM = NU - NZFULL * ZCH
ZREM_TILE = NZFULL % NS

_F32 = jnp.float32


@functools.partial(
    pl.kernel,
    out_type=jax.ShapeDtypeStruct((NN,), _F32),
    mesh=plsc.VectorSubcoreMesh(core_axis_name="c", subcore_axis_name="s"),
    scratch_types=[
        pltpu.VMEM((NCHUNK, CH), jnp.int32),
        pltpu.VMEM((128,), _F32),
        pltpu.VMEM((320,), _F32),
        pltpu.VMEM_SHARED((NU,), _F32),
    ],
)
def _deg_kernel(eidx_hbm, deg_hbm, idx_v, ones_v, zrow_v, acc):
    c = lax.axis_index("c")
    s = lax.axis_index("s")
    pltpu.sync_copy(eidx_hbm.at[c, s], idx_v)

    @pl.loop(0, 128 // 16)
    def _(i):
        ones_v[pl.ds(i * 16, 16)] = jnp.ones((16,), _F32)

    @pl.loop(0, 320 // 16)
    def _(i):
        zrow_v[pl.ds(i * 16, 16)] = jnp.zeros((16,), _F32)

    @pl.when(s < 15)
    def _():
        pltpu.sync_copy(zrow_v, acc.at[pl.ds(s * 320, 320)])

    @pl.when(s == 15)
    def _():
        pltpu.sync_copy(zrow_v.at[pl.ds(0, 200)], acc.at[pl.ds(4800, 200)])

    plsc.subcore_barrier()

    @pl.loop(0, NCHUNK)
    def _(j):
        pltpu.sync_copy(ones_v.at[pl.ds(0, CH)], acc.at[idx_v.at[j]],
                        add=True)

    plsc.subcore_barrier()

    @pl.when(s < 15)
    def _():
        pltpu.sync_copy(acc.at[pl.ds(s * 320, 320)], zrow_v)
        pltpu.sync_copy(zrow_v, deg_hbm.at[pl.ds(c * NU + s * 320, 320)])

    @pl.when(s == 15)
    def _():
        pltpu.sync_copy(acc.at[pl.ds(4800, 200)], zrow_v.at[pl.ds(0, 200)])
        pltpu.sync_copy(zrow_v.at[pl.ds(0, 200)],
                        deg_hbm.at[pl.ds(c * NU + 4800, 200)])


_OUT3 = [jax.ShapeDtypeStruct((NN, FD), _F32) for _ in range(3)]


@functools.partial(
    pl.kernel,
    out_type=_OUT3,
    mesh=plsc.VectorSubcoreMesh(core_axis_name="c", subcore_axis_name="s"),
    scratch_types=[
        pltpu.VMEM((NCHUNK, CH), jnp.int32),
        pltpu.VMEM((NCHUNK, CH), jnp.int32),
        pltpu.VMEM((CH, FD), _F32),
        pltpu.VMEM((CH, FD), _F32),
        pltpu.VMEM((ZCH, FD), _F32),
        pltpu.VMEM_SHARED((NU, FD), _F32),
        pltpu.SemaphoreType.DMA,
        pltpu.SemaphoreType.DMA,
    ],
)
def _spmm_kernel(f0_hbm, f1_hbm, f2_hbm, srcidx_hbm, dstidx_hbm,
                 o0_hbm, o1_hbm, o2_hbm,
                 sidx_v, didx_v, buf0_v, buf1_v, zbuf_v, acc, sem0, sem1):
    c = lax.axis_index("c")
    s = lax.axis_index("s")
    pltpu.sync_copy(srcidx_hbm.at[c, s], sidx_v)
    pltpu.sync_copy(dstidx_hbm.at[c, s], didx_v)

    @pl.loop(0, ZCH)
    def _(i):
        @pl.loop(0, FD // 16)
        def _(j):
            zbuf_v[i, pl.ds(j * 16, 16)] = jnp.zeros((16,), _F32)

    for feat_hbm, out_hbm in ((f0_hbm, o0_hbm), (f1_hbm, o1_hbm),
                              (f2_hbm, o2_hbm)):
        @pl.loop(0, NZFULL)
        def _(k):
            @pl.when(lax.rem(k, NS) == s)
            def _():
                pltpu.sync_copy(zbuf_v, acc.at[pl.ds(k * ZCH, ZCH)])

        @pl.when(s == ZREM_TILE)
        def _():
            pltpu.sync_copy(zbuf_v.at[pl.ds(0, ZREM)],
                            acc.at[pl.ds(NZFULL * ZCH, ZREM)])

        plsc.subcore_barrier()

        pltpu.async_copy(feat_hbm.at[sidx_v.at[0]], buf0_v, sem0)

        @pl.loop(0, NCHUNK // 2 - 1)
        def _(k):
            a = 2 * k
            b = a + 1
            pltpu.make_async_copy(feat_hbm.at[sidx_v.at[a]], buf0_v,
                                  sem0).wait()
            pltpu.async_copy(feat_hbm.at[sidx_v.at[b]], buf1_v, sem1)
            pltpu.sync_copy(buf0_v, acc.at[didx_v.at[a]], add=True)
            pltpu.make_async_copy(feat_hbm.at[sidx_v.at[b]], buf1_v,
                                  sem1).wait()
            pltpu.async_copy(feat_hbm.at[sidx_v.at[a + 2]], buf0_v, sem0)
            pltpu.sync_copy(buf1_v, acc.at[didx_v.at[b]], add=True)

        pltpu.make_async_copy(feat_hbm.at[sidx_v.at[NCHUNK - 2]], buf0_v,
                              sem0).wait()
        pltpu.async_copy(feat_hbm.at[sidx_v.at[NCHUNK - 1]], buf1_v, sem1)
        pltpu.sync_copy(buf0_v, acc.at[didx_v.at[NCHUNK - 2]], add=True)
        pltpu.make_async_copy(feat_hbm.at[sidx_v.at[NCHUNK - 1]], buf1_v,
                              sem1).wait()
        pltpu.sync_copy(buf1_v, acc.at[didx_v.at[NCHUNK - 1]], add=True)

        plsc.subcore_barrier()

        @pl.loop(0, NZFULL)
        def _(k):
            @pl.when(lax.rem(k, NS) == s)
            def _():
                pltpu.sync_copy(acc.at[pl.ds(k * ZCH, ZCH)],
                                buf0_v.at[pl.ds(0, ZCH)])
                pltpu.sync_copy(buf0_v.at[pl.ds(0, ZCH)],
                                out_hbm.at[pl.ds(c * NU + k * ZCH, ZCH)])

        @pl.when(s == ZREM_TILE)
        def _():
            pltpu.sync_copy(acc.at[pl.ds(NZFULL * ZCH, ZREM)],
                            buf0_v.at[pl.ds(0, ZREM)])
            pltpu.sync_copy(buf0_v.at[pl.ds(0, ZREM)],
                            out_hbm.at[pl.ds(c * NU + NZFULL * ZCH, ZREM)])

        plsc.subcore_barrier()


_RB = 2000


def _prep_body(deg_ref, x_ref, r1_ref, r2_ref, f0_ref, f1_ref, f2_ref):
    deg = deg_ref[...]
    d_inv = jnp.where(deg > 0.0, lax.rsqrt(deg), 0.0)
    xb = x_ref[...]
    sx = jnp.sign(xb)
    p1 = xb + r1_ref[...] * sx * EPS
    p2 = xb + r2_ref[...] * sx * EPS
    f0_ref[...] = xb * d_inv
    f1_ref[...] = p1 * d_inv
    f2_ref[...] = p2 * d_inv


_prep = pl.pallas_call(
    _prep_body,
    grid=(NN // _RB,),
    in_specs=[
        pl.BlockSpec((_RB, 1), lambda i: (i, 0)),
        pl.BlockSpec((_RB, FD), lambda i: (i, 0)),
        pl.BlockSpec((_RB, FD), lambda i: (i, 0)),
        pl.BlockSpec((_RB, FD), lambda i: (i, 0)),
    ],
    out_specs=[
        pl.BlockSpec((_RB, FD), lambda i: (i, 0)),
        pl.BlockSpec((_RB, FD), lambda i: (i, 0)),
        pl.BlockSpec((_RB, FD), lambda i: (i, 0)),
    ],
    out_shape=[
        jax.ShapeDtypeStruct((NN, FD), _F32),
        jax.ShapeDtypeStruct((NN, FD), _F32),
        jax.ShapeDtypeStruct((NN, FD), _F32),
    ],
)


def _post_body(deg_ref, x_ref, r1_ref, r2_ref, a0_ref, a1_ref, a2_ref,
               alle_ref, allp1_ref, allp2_ref, g0_ref):
    deg = deg_ref[...]
    d_inv = jnp.where(deg > 0.0, lax.rsqrt(deg), 0.0)
    xb = x_ref[...]
    sx = jnp.sign(xb)
    p1 = xb + r1_ref[...] * sx * EPS
    p2 = xb + r2_ref[...] * sx * EPS
    g0 = a0_ref[...] * d_inv
    g1 = a1_ref[...] * d_inv
    g2 = a2_ref[...] * d_inv
    alle_ref[...] = 2.0 * xb + g0
    allp1_ref[...] = 2.0 * p1 + g1
    allp2_ref[...] = 2.0 * p2 + g2
    g0_ref[...] = g0


_post = pl.pallas_call(
    _post_body,
    grid=(NN // _RB,),
    in_specs=[
        pl.BlockSpec((_RB, 1), lambda i: (i, 0)),
        pl.BlockSpec((_RB, FD), lambda i: (i, 0)),
        pl.BlockSpec((_RB, FD), lambda i: (i, 0)),
        pl.BlockSpec((_RB, FD), lambda i: (i, 0)),
        pl.BlockSpec((_RB, FD), lambda i: (i, 0)),
        pl.BlockSpec((_RB, FD), lambda i: (i, 0)),
        pl.BlockSpec((_RB, FD), lambda i: (i, 0)),
    ],
    out_specs=[
        pl.BlockSpec((_RB, FD), lambda i: (i, 0)),
        pl.BlockSpec((_RB, FD), lambda i: (i, 0)),
        pl.BlockSpec((_RB, FD), lambda i: (i, 0)),
        pl.BlockSpec((_RB, FD), lambda i: (i, 0)),
    ],
    out_shape=[
        jax.ShapeDtypeStruct((NN, FD), _F32),
        jax.ShapeDtypeStruct((NN, FD), _F32),
        jax.ShapeDtypeStruct((NN, FD), _F32),
        jax.ShapeDtypeStruct((NN, FD), _F32),
    ],
)


def kernel(x, rand1, rand2, edge_u, edge_i):
    eu = edge_u.astype(jnp.int32)
    ei = edge_i.astype(jnp.int32)
    dst_idx = jnp.stack([eu, ei]).reshape(2, NS, NCHUNK, CH)
    src_idx = jnp.stack([ei + NU, eu]).reshape(2, NS, NCHUNK, CH)

    deg = _deg_kernel(dst_idx)
    deg2 = deg.reshape(NN, 1)
    f0, f1, f2 = _prep(deg2, x, rand1, rand2)
    a0, a1, a2 = _spmm_kernel(f0, f1, f2, src_idx, dst_idx)
    all_e, all_p1, all_p2, g0 = _post(deg2, x, rand1, rand2, a0, a1, a2)
    return (all_e[:NU], all_e[NU:], all_p1[:NU], all_p1[NU:],
            all_p2[:NU], all_p2[NU:], g0)
